# Initial kernel scaffold; baseline (speedup 1.0000x reference)
import jax
import jax.numpy as jnp
from jax.experimental import pallas as pl


def kernel(positions, tables):
    raise NotImplementedError("write your pallas kernel here")


# speedup vs baseline: 64.6893x; 64.6893x over previous
import functools

import jax
import jax.numpy as jnp
import numpy as np
from jax import lax
from jax.experimental import pallas as pl
from jax.experimental.pallas import tpu as pltpu
from jax.experimental.pallas import tpu_sc as plsc

NUM_LEVELS = 16
FPL = 2
HASHMAP = 2 ** 19
BASE = 16
FINEST = 512
RES = [int(np.floor(BASE * np.exp(i * np.log(FINEST / BASE) / (NUM_LEVELS - 1))))
       for i in range(NUM_LEVELS)]
P2 = 2654435761
P3 = 805459861
MASK = HASHMAP - 1

NC = 2
NS = 16
NW = NC * NS
L = 16

CHUNK = 1024
GROUPS = CHUNK // L


def _body(px_hbm, py_hbm, pz_hbm, tab_hbm, out_hbm,
          posx, posy, posz,
          w0, w1, idx0, idx1, g0, g1, obuf, sem0, sem1, npad):
    cid = lax.axis_index("c")
    sid = lax.axis_index("s")
    wid = sid * NC + cid
    per_w = npad // NW
    n_chunks = per_w // CHUNK
    tile_base = wid * per_w

    wbufs = (w0, w1)
    idxbufs = (idx0, idx1)
    gbufs = (g0, g1)
    sems = (sem0, sem1)

    def chunk_body(ci, carry):
        base = tile_base + ci * CHUNK
        pltpu.sync_copy(px_hbm.at[pl.ds(base, CHUNK)], posx)
        pltpu.sync_copy(py_hbm.at[pl.ds(base, CHUNK)], posy)
        pltpu.sync_copy(pz_hbm.at[pl.ds(base, CHUNK)], posz)

        def axis_terms(pref, j0, res):
            p = pref[pl.ds(j0, L)]
            s = ((p + 1.0) * 0.5) * res
            i = s.astype(jnp.int32)
            fr = s - i.astype(jnp.float32)
            return i.astype(jnp.uint32), fr

        def phase_a(li):
            par = li % 2
            wb = wbufs[par]
            ib = idxbufs[par]
            res = float(RES[li])
            lbase = jnp.uint32(li * HASHMAP)

            def grp(t, c):
                j0 = t * L
                xu, fx = axis_terms(posx, j0, res)
                yu, fy = axis_terms(posy, j0, res)
                zu, fz = axis_terms(posz, j0, res)
                ax0 = xu
                ax1 = xu + jnp.uint32(1)
                by0 = yu * jnp.uint32(P2)
                by1 = by0 + jnp.uint32(P2)
                cz0 = zu * jnp.uint32(P3)
                cz1 = cz0 + jnp.uint32(P3)
                bc = (by0 ^ cz0, by0 ^ cz1, by1 ^ cz0, by1 ^ cz1)
                wxv = (1.0 - fx, fx)
                wyv = (1.0 - fy, fy)
                wzv = (1.0 - fz, fz)
                for k in range(8):
                    dx, dy, dz = k >> 2, (k >> 1) & 1, k & 1
                    ax = ax1 if dx else ax0
                    h = ((ax ^ bc[dy * 2 + dz]) & jnp.uint32(MASK)) | lbase
                    ib[pl.ds(k * CHUNK + j0, L)] = h.astype(jnp.int32)
                    wk = (wxv[dx] * wyv[dy]) * wzv[dz]
                    wb[k, pl.ds(j0, L)] = wk
                return c

            lax.fori_loop(0, GROUPS, grp, 0)
            return pltpu.async_copy(tab_hbm.at[ib], gbufs[par], sems[par])

        def phase_b(li, desc):
            desc.wait()
            par = li % 2
            wb = wbufs[par]
            gb = gbufs[par]

            def grp(t, c):
                j0 = t * L
                acc0 = jnp.zeros((L,), jnp.float32)
                acc1 = jnp.zeros((L,), jnp.float32)
                for k in range(8):
                    w = gb[pl.ds(k * CHUNK + j0, L)]
                    f0 = lax.bitcast_convert_type(w << 16, jnp.float32)
                    f1 = lax.bitcast_convert_type(w & jnp.int32(-65536), jnp.float32)
                    wk = wb[k, pl.ds(j0, L)]
                    acc0 = acc0 + wk * f0
                    acc1 = acc1 + wk * f1
                obuf[2 * li, pl.ds(j0, L)] = acc0
                obuf[2 * li + 1, pl.ds(j0, L)] = acc1
                return c

            lax.fori_loop(0, GROUPS, grp, 0)

        descs = [None, None]
        descs[0] = phase_a(0)
        for li in range(1, NUM_LEVELS):
            descs[li % 2] = phase_a(li)
            phase_b(li - 1, descs[(li - 1) % 2])
        phase_b(NUM_LEVELS - 1, descs[(NUM_LEVELS - 1) % 2])

        pltpu.sync_copy(obuf, out_hbm.at[wid * n_chunks + ci])
        return carry

    lax.fori_loop(0, n_chunks, chunk_body, 0)


@functools.partial(jax.jit, static_argnames=("npad",))
def _run(px, py, pz, tabp, npad):
    mesh = plsc.VectorSubcoreMesh(core_axis_name="c", subcore_axis_name="s")
    kfn = pl.kernel(
        functools.partial(_body, npad=npad),
        out_type=jax.ShapeDtypeStruct(
            (npad // CHUNK, 2 * NUM_LEVELS, CHUNK), jnp.float32),
        mesh=mesh,
        scratch_types=[
            pltpu.VMEM((CHUNK,), jnp.float32),
            pltpu.VMEM((CHUNK,), jnp.float32),
            pltpu.VMEM((CHUNK,), jnp.float32),
            pltpu.VMEM((8, CHUNK), jnp.float32),
            pltpu.VMEM((8, CHUNK), jnp.float32),
            pltpu.VMEM((8 * CHUNK,), jnp.int32),
            pltpu.VMEM((8 * CHUNK,), jnp.int32),
            pltpu.VMEM((8 * CHUNK,), jnp.int32),
            pltpu.VMEM((8 * CHUNK,), jnp.int32),
            pltpu.VMEM((2 * NUM_LEVELS, CHUNK), jnp.float32),
            pltpu.SemaphoreType.DMA,
            pltpu.SemaphoreType.DMA,
        ],
    )
    return kfn(px, py, pz, tabp)


def kernel(positions, tables):
    n = positions.shape[0]
    npad = -(-n // (NW * CHUNK)) * (NW * CHUNK)
    pos = jnp.pad(positions, ((0, npad - n), (0, 0)))
    px, py, pz = pos[:, 0], pos[:, 1], pos[:, 2]
    tb = jax.lax.bitcast_convert_type(
        tables.astype(jnp.bfloat16), jnp.uint16).astype(jnp.uint32)
    word = tb[..., 0] | (tb[..., 1] << 16)
    tabp = jax.lax.bitcast_convert_type(
        word.reshape(NUM_LEVELS * HASHMAP), jnp.int32)
    out = _run(px, py, pz, tabp, npad)
    return out.transpose(0, 2, 1).reshape(npad, 2 * NUM_LEVELS)[:n]


# speedup vs baseline: 64.7121x; 1.0004x over previous
import functools

import jax
import jax.numpy as jnp
import numpy as np
from jax import lax
from jax.experimental import pallas as pl
from jax.experimental.pallas import tpu as pltpu
from jax.experimental.pallas import tpu_sc as plsc

NUM_LEVELS = 16
FPL = 2
HASHMAP = 2 ** 19
BASE = 16
FINEST = 512
RES = [int(np.floor(BASE * np.exp(i * np.log(FINEST / BASE) / (NUM_LEVELS - 1))))
       for i in range(NUM_LEVELS)]
P2 = 2654435761
P3 = 805459861
MASK = HASHMAP - 1

NC = 2
NS = 16
NW = NC * NS
L = 16

CHUNK = 1024
GROUPS = CHUNK // L


def _body(px_hbm, py_hbm, pz_hbm, tab_hbm, out_hbm,
          posx, posy, posz,
          w0, w1, w2, idx0, idx1, idx2, g0, g1, g2, obuf,
          sem0, sem1, sem2, npad):
    cid = lax.axis_index("c")
    sid = lax.axis_index("s")
    wid = sid * NC + cid
    per_w = npad // NW
    n_chunks = per_w // CHUNK
    tile_base = wid * per_w

    wbufs = (w0, w1, w2)
    idxbufs = (idx0, idx1, idx2)
    gbufs = (g0, g1, g2)
    sems = (sem0, sem1, sem2)

    def chunk_body(ci, carry):
        base = tile_base + ci * CHUNK
        pltpu.sync_copy(px_hbm.at[pl.ds(base, CHUNK)], posx)
        pltpu.sync_copy(py_hbm.at[pl.ds(base, CHUNK)], posy)
        pltpu.sync_copy(pz_hbm.at[pl.ds(base, CHUNK)], posz)

        def axis_terms(pref, j0, res):
            p = pref[pl.ds(j0, L)]
            s = ((p + 1.0) * 0.5) * res
            i = s.astype(jnp.int32)
            fr = s - i.astype(jnp.float32)
            return i.astype(jnp.uint32), fr

        def emit_a(li, t):
            par = li % 3
            wb = wbufs[par]
            ib = idxbufs[par]
            res = float(RES[li])
            lbase = jnp.uint32(li * HASHMAP)
            j0 = t * L
            xu, fx = axis_terms(posx, j0, res)
            yu, fy = axis_terms(posy, j0, res)
            zu, fz = axis_terms(posz, j0, res)
            ax0 = xu
            ax1 = xu + jnp.uint32(1)
            by0 = yu * jnp.uint32(P2)
            by1 = by0 + jnp.uint32(P2)
            cz0 = zu * jnp.uint32(P3)
            cz1 = cz0 + jnp.uint32(P3)
            bc = (by0 ^ cz0, by0 ^ cz1, by1 ^ cz0, by1 ^ cz1)
            wxv = (1.0 - fx, fx)
            wyv = (1.0 - fy, fy)
            wzv = (1.0 - fz, fz)
            for k in range(8):
                dx, dy, dz = k >> 2, (k >> 1) & 1, k & 1
                ax = ax1 if dx else ax0
                h = ((ax ^ bc[dy * 2 + dz]) & jnp.uint32(MASK)) | lbase
                ib[pl.ds(k * CHUNK + j0, L)] = h.astype(jnp.int32)
                wk = (wxv[dx] * wyv[dy]) * wzv[dz]
                wb[k, pl.ds(j0, L)] = wk

        def emit_b(li, t):
            par = li % 3
            wb = wbufs[par]
            gb = gbufs[par]
            j0 = t * L
            acc0 = jnp.zeros((L,), jnp.float32)
            acc1 = jnp.zeros((L,), jnp.float32)
            for k in range(8):
                w = gb[pl.ds(k * CHUNK + j0, L)]
                f0 = lax.bitcast_convert_type(w << 16, jnp.float32)
                f1 = lax.bitcast_convert_type(w & jnp.int32(-65536), jnp.float32)
                wk = wb[k, pl.ds(j0, L)]
                acc0 = acc0 + wk * f0
                acc1 = acc1 + wk * f1
            obuf[2 * li, pl.ds(j0, L)] = acc0
            obuf[2 * li + 1, pl.ds(j0, L)] = acc1

        def start(li):
            return pltpu.async_copy(
                tab_hbm.at[idxbufs[li % 3]], gbufs[li % 3], sems[li % 3])

        def loop_a(li):
            def grp(t, c):
                emit_a(li, t)
                return c
            lax.fori_loop(0, GROUPS, grp, 0)

        def loop_ab(li_a, li_b):
            def grp(t, c):
                emit_a(li_a, t)
                emit_b(li_b, t)
                return c
            lax.fori_loop(0, GROUPS, grp, 0)

        def loop_b(li):
            def grp(t, c):
                emit_b(li, t)
                return c
            lax.fori_loop(0, GROUPS, grp, 0)

        descs = [None, None, None]
        loop_a(0)
        descs[0] = start(0)
        loop_a(1)
        descs[1] = start(1)
        for li in range(NUM_LEVELS):
            descs[li % 3].wait()
            if li + 2 < NUM_LEVELS:
                loop_ab(li + 2, li)
                descs[(li + 2) % 3] = start(li + 2)
            else:
                loop_b(li)

        pltpu.sync_copy(obuf, out_hbm.at[wid * n_chunks + ci])
        return carry

    lax.fori_loop(0, n_chunks, chunk_body, 0)


@functools.partial(jax.jit, static_argnames=("npad",))
def _run(px, py, pz, tabp, npad):
    mesh = plsc.VectorSubcoreMesh(core_axis_name="c", subcore_axis_name="s")
    kfn = pl.kernel(
        functools.partial(_body, npad=npad),
        out_type=jax.ShapeDtypeStruct(
            (npad // CHUNK, 2 * NUM_LEVELS, CHUNK), jnp.float32),
        mesh=mesh,
        scratch_types=[
            pltpu.VMEM((CHUNK,), jnp.float32),
            pltpu.VMEM((CHUNK,), jnp.float32),
            pltpu.VMEM((CHUNK,), jnp.float32),
            pltpu.VMEM((8, CHUNK), jnp.float32),
            pltpu.VMEM((8, CHUNK), jnp.float32),
            pltpu.VMEM((8, CHUNK), jnp.float32),
            pltpu.VMEM((8 * CHUNK,), jnp.int32),
            pltpu.VMEM((8 * CHUNK,), jnp.int32),
            pltpu.VMEM((8 * CHUNK,), jnp.int32),
            pltpu.VMEM((8 * CHUNK,), jnp.int32),
            pltpu.VMEM((8 * CHUNK,), jnp.int32),
            pltpu.VMEM((8 * CHUNK,), jnp.int32),
            pltpu.VMEM((2 * NUM_LEVELS, CHUNK), jnp.float32),
            pltpu.SemaphoreType.DMA,
            pltpu.SemaphoreType.DMA,
            pltpu.SemaphoreType.DMA,
        ],
    )
    return kfn(px, py, pz, tabp)


def kernel(positions, tables):
    n = positions.shape[0]
    npad = -(-n // (NW * CHUNK)) * (NW * CHUNK)
    pos = jnp.pad(positions, ((0, npad - n), (0, 0)))
    px, py, pz = pos[:, 0], pos[:, 1], pos[:, 2]
    tb = jax.lax.bitcast_convert_type(
        tables.astype(jnp.bfloat16), jnp.uint16).astype(jnp.uint32)
    word = tb[..., 0] | (tb[..., 1] << 16)
    tabp = jax.lax.bitcast_convert_type(
        word.reshape(NUM_LEVELS * HASHMAP), jnp.int32)
    out = _run(px, py, pz, tabp, npad)
    return out.transpose(0, 2, 1).reshape(npad, 2 * NUM_LEVELS)[:n]
